# baseline (speedup 1.0000x reference)
import functools

import jax
import jax.numpy as jnp
from jax import lax
from jax.experimental import pallas as pl
from jax.experimental.pallas import tpu as pltpu
from jax.experimental.pallas import tpu_sc as plsc

_NR = 10000
_NI = 10000
_ND = 5000
_D = 128
_DE = 16
_L = 2
_ALPHA = 0.1

_NRP = 10240
_NDP = 5120
_NC = 2
_NS = 16
_NW = _NC * _NS
_CHUNK = 128

_LN_ALPHA = -2.302585092994046


_NBUF = 4
_GS = 16


def _edge_segsum(n_out_pad, e_pad, mode, chunk=_CHUNK, d=_D):
  per_core = mode == "pm"
  et = e_pad // (_NS if per_core else _NW)
  nchunk = et // chunk
  ngrp = nchunk // _GS
  rpt = n_out_pad // _NS

  scratch = [
      pltpu.VMEM((_GS, chunk), jnp.int32),
      pltpu.VMEM((_GS, chunk), jnp.int32),
  ]
  nrows = 1 if mode == "deg" else _NBUF
  scratch += [pltpu.VMEM((chunk, d), jnp.float32) for _ in range(nrows)]
  if mode == "msg":
    scratch += [pltpu.VMEM((chunk, d), jnp.float32) for _ in range(_NBUF)]
  nsem = 2 * _NBUF + (_NBUF if mode == "msg" else 0)
  scratch += [pltpu.SemaphoreType.DMA for _ in range(nsem)]
  scratch.append(pltpu.VMEM_SHARED((n_out_pad, d), jnp.float32))
  mesh = plsc.VectorSubcoreMesh(core_axis_name="c", subcore_axis_name="s")
  out_type = jax.ShapeDtypeStruct((_NC, n_out_pad, d), jnp.float32)

  def body(*refs):
    if mode == "msg":
      table, src2, dst2, ew, zrows, out = refs[:6]
      sc = refs[6:]
      srcb, dstb = sc[0], sc[1]
      rows = sc[2:2 + _NBUF]
      ewb = sc[2 + _NBUF:2 + 2 * _NBUF]
      gsem = sc[2 + 2 * _NBUF:2 + 3 * _NBUF]
      ssem = sc[2 + 3 * _NBUF:2 + 4 * _NBUF]
      esem = sc[2 + 4 * _NBUF:2 + 5 * _NBUF]
      acc = sc[-1]
    elif mode == "pm":
      tab0, tab1, src2, dst2, zrows, out = refs[:6]
      sc = refs[6:]
      srcb, dstb = sc[0], sc[1]
      rows = sc[2:2 + _NBUF]
      gsem = sc[2 + _NBUF:2 + 2 * _NBUF]
      ssem = sc[2 + 2 * _NBUF:2 + 3 * _NBUF]
      acc = sc[-1]
    elif mode == "deg":
      dst2, zrows, out = refs[:3]
      sc = refs[3:]
      srcb, dstb = sc[0], sc[1]
      rows = sc[2:3]
      gsem = sc[3:3 + _NBUF]
      ssem = sc[3 + _NBUF:3 + 2 * _NBUF]
      acc = sc[-1]
    else:
      table, src2, dst2, zrows, out = refs[:5]
      sc = refs[5:]
      srcb, dstb = sc[0], sc[1]
      rows = sc[2:2 + _NBUF]
      gsem = sc[2 + _NBUF:2 + 2 * _NBUF]
      ssem = sc[2 + 2 * _NBUF:2 + 3 * _NBUF]
      acc = sc[-1]
    cid = lax.axis_index("c")
    sid = lax.axis_index("s")
    wid = sid * _NC + cid
    row0 = sid * rpt
    pltpu.sync_copy(zrows.at[pl.ds(row0, rpt)], acc.at[pl.ds(row0, rpt)])
    plsc.subcore_barrier()

    if mode == "deg":
      def init_ones(i, c0):
        rows[0][i, pl.ds(0, 16)] = jnp.full((16,), 1.0, jnp.float32)
        return c0
      lax.fori_loop(0, chunk, init_ones, 0)

    def make_group(table):
      def group(g, carry):
        base = (sid if per_core else wid) * nchunk + g * _GS
        if mode != "deg":
          pltpu.sync_copy(src2.at[pl.ds(base, _GS)], srcb)
        pltpu.sync_copy(dst2.at[pl.ds(base, _GS)], dstb)

        def issue(b, j):
          if mode != "deg":
            pltpu.async_copy(table.at[srcb.at[j]], rows[b], gsem[b])
          if mode == "msg":
            pltpu.async_copy(ew.at[pl.ds((base + j) * chunk, chunk)],
                             ewb[b], esem[b])

        for b in range(_NBUF):
          issue(b, b)
        for half in range(_GS // _NBUF):
          for b in range(_NBUF):
            j = half * _NBUF + b
            if mode != "deg":
              pltpu.make_async_copy(table.at[srcb.at[j]], rows[b],
                                    gsem[b]).wait()
            if mode == "msg":
              pltpu.make_async_copy(ew.at[pl.ds((base + j) * chunk, chunk)],
                                    ewb[b], esem[b]).wait()
              rb, eb = rows[b], ewb[b]

              def ebody(i, c2, rb=rb, eb=eb):
                for kk in range(d // 16):
                  sl = pl.ds(kk * 16, 16)
                  rb[i, sl] = jnp.maximum(rb[i, sl] + eb[i, sl], 0.0)
                return c2

              lax.fori_loop(0, chunk, ebody, 0)
            rsrc = rows[0] if mode == "deg" else rows[b]
            pltpu.async_copy(rsrc, acc.at[dstb.at[j]], ssem[b], add=True)
          for b in range(_NBUF):
            j = half * _NBUF + b
            rsrc = rows[0] if mode == "deg" else rows[b]
            pltpu.make_async_copy(rsrc, acc.at[dstb.at[j]], ssem[b]).wait()
            if half < _GS // _NBUF - 1:
              issue(b, (half + 1) * _NBUF + b)
        return carry
      return group

    if mode == "pm":
      @pl.when(cid == 0)
      def _():
        lax.fori_loop(0, ngrp, make_group(tab0), 0)

      @pl.when(cid == 1)
      def _():
        lax.fori_loop(0, ngrp, make_group(tab1), 0)
    else:
      lax.fori_loop(0, ngrp, make_group(None if mode == "deg" else table), 0)
    plsc.subcore_barrier()
    pltpu.sync_copy(acc.at[pl.ds(row0, rpt)], out.at[cid, pl.ds(row0, rpt)])

  return pl.kernel(body, out_type=out_type, mesh=mesh, scratch_types=scratch)


def _mm(a, b):
  return lax.dot_general(a, b, (((1,), (0,)), ((), ())),
                         preferred_element_type=jnp.float32)


def _tc_linear(x, w, b, bs):
  n, k = x.shape
  m = w.shape[1]

  def body(x_ref, w_ref, b_ref, o_ref):
    o_ref[...] = _mm(x_ref[...], w_ref[...]) + b_ref[...]

  return pl.pallas_call(
      body, grid=(n // bs,),
      in_specs=[pl.BlockSpec((bs, k), lambda i: (i, 0)),
                pl.BlockSpec((k, m), lambda i: (0, 0)),
                pl.BlockSpec((1, m), lambda i: (0, 0))],
      out_specs=pl.BlockSpec((bs, m), lambda i: (i, 0)),
      out_shape=jax.ShapeDtypeStruct((n, m), jnp.float32),
  )(x, w, b)


def _tc_update(agg, w, b, res, bs):
  n = res.shape[0]

  def body(a_ref, w_ref, b_ref, r_ref, o_ref):
    x = a_ref[0] + a_ref[1]
    o_ref[...] = jnp.maximum(_mm(x, w_ref[...]) + b_ref[...], 0.0) + r_ref[...]

  return pl.pallas_call(
      body, grid=(n // bs,),
      in_specs=[pl.BlockSpec((2, bs, _D), lambda i: (0, i, 0)),
                pl.BlockSpec((_D, _D), lambda i: (0, 0)),
                pl.BlockSpec((1, _D), lambda i: (0, 0)),
                pl.BlockSpec((bs, _D), lambda i: (i, 0))],
      out_specs=pl.BlockSpec((bs, _D), lambda i: (i, 0)),
      out_shape=jax.ShapeDtypeStruct((n, _D), jnp.float32),
  )(agg, w, b, res)


def _tc_gcn2(agg2, deg2, hp, hm, wp, bp, wm, bm, bs):
  n = hp.shape[0]

  def body(a_ref, d_ref, hp_ref, hm_ref, wp_ref, bp_ref, wm_ref, bm_ref,
           op_ref, om_ref):
    inv = 1.0 / jnp.maximum(d_ref[0, :, :1] + d_ref[1, :, :1], 1.0)
    xp = a_ref[0] * inv
    xm = a_ref[1] * inv
    op_ref[...] = jnp.maximum(_mm(xp, wp_ref[...]) + bp_ref[...], 0.0) + hp_ref[...]
    om_ref[...] = jnp.maximum(_mm(xm, wm_ref[...]) + bm_ref[...], 0.0) + hm_ref[...]

  s2 = pl.BlockSpec((2, bs, _D), lambda i: (0, i, 0))
  s2d = pl.BlockSpec((2, bs, _D), lambda i: (0, i, 0))
  sd = pl.BlockSpec((bs, _D), lambda i: (i, 0))
  sw = pl.BlockSpec((_D, _D), lambda i: (0, 0))
  sb = pl.BlockSpec((1, _D), lambda i: (0, 0))
  sh = jax.ShapeDtypeStruct((n, _D), jnp.float32)
  return pl.pallas_call(
      body, grid=(n // bs,),
      in_specs=[s2, s2d, sd, sd, sw, sb, sw, sb],
      out_specs=[sd, sd], out_shape=[sh, sh],
  )(agg2, deg2, hp, hm, wp, bp, wm, bm)


def _tc_update2(agg2, wp, bp, wm, bm, rp, rm, bs):
  n = rp.shape[0]

  def body(a_ref, wp_ref, bp_ref, wm_ref, bm_ref, rp_ref, rm_ref,
           op_ref, om_ref):
    op_ref[...] = jnp.maximum(_mm(a_ref[0], wp_ref[...]) + bp_ref[...], 0.0) + rp_ref[...]
    om_ref[...] = jnp.maximum(_mm(a_ref[1], wm_ref[...]) + bm_ref[...], 0.0) + rm_ref[...]

  s2 = pl.BlockSpec((2, bs, _D), lambda i: (0, i, 0))
  sd = pl.BlockSpec((bs, _D), lambda i: (i, 0))
  sw = pl.BlockSpec((_D, _D), lambda i: (0, 0))
  sb = pl.BlockSpec((1, _D), lambda i: (0, 0))
  sh = jax.ShapeDtypeStruct((n, _D), jnp.float32)
  return pl.pallas_call(
      body, grid=(n // bs,),
      in_specs=[s2, sw, sb, sw, sb, sd, sd],
      out_specs=[sd, sd], out_shape=[sh, sh],
  )(agg2, wp, bp, wm, bm, rp, rm)


def _tc_add2(a, b, r, bs):
  n = r.shape[0]

  def body(a_ref, b_ref, r_ref, o1_ref, o2_ref):
    o1_ref[...] = a_ref[...] + r_ref[...]
    o2_ref[...] = b_ref[...] + r_ref[...]

  sp = pl.BlockSpec((bs, _D), lambda i: (i, 0))
  sh = jax.ShapeDtypeStruct((n, _D), jnp.float32)
  return pl.pallas_call(
      body, grid=(n // bs,),
      in_specs=[sp, sp, sp], out_specs=[sp, sp], out_shape=[sh, sh],
  )(a, b, r)


def _tc_scale4(d, bs):
  n = d.shape[0]
  nb = n // bs

  def body(d_ref, o_ref):
    k = (pl.program_id(0) // nb).astype(jnp.float32)
    o_ref[...] = d_ref[...] * jnp.exp(k * _LN_ALPHA)

  return pl.pallas_call(
      body, grid=(4 * nb,),
      in_specs=[pl.BlockSpec((bs, _D), lambda j: (j % nb, 0))],
      out_specs=pl.BlockSpec((bs, _D), lambda j: (j, 0)),
      out_shape=jax.ShapeDtypeStruct((4 * n, _D), jnp.float32),
  )(d)


def _leaky(x):
  return jnp.where(x >= 0.0, x, 0.01 * x)


def _tc_head(x, w1, b1, w2p, b2p, wplt, bpl, g, bet):
  n = x.shape[0]

  def body(x_ref, w1_ref, b1_ref, w2_ref, b2_ref, wp_ref, bp_ref,
           g_ref, be_ref, o_ref):
    s1 = _leaky(_mm(x_ref[...], w1_ref[...]) + b1_ref[...])
    s2 = _leaky(_mm(s1, w2_ref[...]) + b2_ref[...])
    s3 = jnp.sum(s2 * wp_ref[...], axis=1, keepdims=True) + bp_ref[...]
    mask = lax.broadcasted_iota(jnp.int32, (n, 1), 0) < _ND
    cnt = jnp.float32(_ND)
    mean = jnp.sum(jnp.where(mask, s3, 0.0)) / cnt
    dev = jnp.where(mask, s3 - mean, 0.0)
    var = jnp.sum(dev * dev) / cnt
    s = (s3 - mean) * lax.rsqrt(var + 1e-5) * g_ref[...] + be_ref[...]
    o_ref[...] = 1.0 / (1.0 + jnp.exp(-s))

  full = lambda shape: pl.BlockSpec(shape, lambda: tuple(0 for _ in shape))
  return pl.pallas_call(
      body,
      in_specs=[full((n, _D)), full((_D, _D // 2)), full((1, _D // 2)),
                full((_D // 2, _D)), full((1, _D)), full((1, _D)),
                full((1, 1)), full((1, 1)), full((1, 1))],
      out_specs=full((n, 1)),
      out_shape=jax.ShapeDtypeStruct((n, 1), jnp.float32),
  )(x, w1, b1, w2p, b2p, wplt, bpl, g, bet)


def _tc_final(ss2, r, bs):
  n = r.shape[0]

  def body(s_ref, r_ref, o1_ref, o2_ref):
    o1_ref[...] = s_ref[0, :, :1] * r_ref[...]
    o2_ref[...] = s_ref[1, :, :1] * r_ref[...]

  s32 = pl.BlockSpec((2, bs, _D), lambda i: (0, i, 0))
  sd = pl.BlockSpec((bs, _D), lambda i: (i, 0))
  sh = jax.ShapeDtypeStruct((n, _D), jnp.float32)
  return pl.pallas_call(
      body, grid=(n // bs,),
      in_specs=[s32, sd], out_specs=[sd, sd], out_shape=[sh, sh],
  )(ss2, r)


def _pad_ei(ei, dummy, chunk):
  e = ei.shape[1]
  ep = -(-e // (_NW * chunk * _GS)) * (_NW * chunk * _GS)
  src = jnp.pad(ei[0].astype(jnp.int32), (0, ep - e))
  dst = jnp.pad(ei[1].astype(jnp.int32), (0, ep - e), constant_values=dummy)
  return src.reshape(ep // chunk, chunk), dst.reshape(ep // chunk, chunk), ep


def kernel(r_node, r2r_edge, i_node, d2d_edge, r2r_ei, i2i_ei, d2d_ei, i2d_ei,
           d2r_ei, W_msg, b_msg, W_upd, b_upd, W_iP, b_iP, W_iM, b_iM, W_dP,
           b_dP, W_dM, b_dM, W1P, b1P, W2P, b2P, WplP, bplP, W1M, b1M, W2M,
           b2M, WplM, bplM, gP, betP, gM, betM):
  f32 = jnp.float32

  r = jnp.pad(r_node, ((0, _NRP - _NR), (0, 0)))
  i0 = jnp.pad(i_node, ((0, _NRP - _NI), (0, 0)))

  srcR, dstR, epR = _pad_ei(r2r_ei, _NR, 32)
  srcI, dstI, epI = _pad_ei(i2i_ei, _NI, 64)
  srcID, dstID, epID = _pad_ei(i2d_ei, _ND, 128)
  srcDD, dstDD, epDD = _pad_ei(d2d_ei, _ND, 128)
  srcDR, dstDR, epDR = _pad_ei(d2r_ei, _NR, 64)
  distp = jnp.pad(d2d_edge.astype(jnp.int32), (0, epDD - d2d_edge.shape[0]))
  srcDD = (srcDD.reshape(-1) + distp * _NDP).reshape(-1, 128)
  e16 = jnp.pad(r2r_edge, ((0, epR - r2r_edge.shape[0]), (0, 0)))

  zR = jnp.zeros((_NRP, _D), f32)
  zD = jnp.zeros((_NDP, _D), f32)

  seg_r2r = _edge_segsum(_NRP, epR, "msg", 32)
  seg_i2i = _edge_segsum(_NRP, epI, "pm", 64)
  seg_i2d = _edge_segsum(_NDP, epID, "pm", 128)
  seg_d2d = _edge_segsum(_NDP, epDD, "pm", 128)
  seg_d2r = _edge_segsum(_NRP, epDR, "pm", 64)
  seg_deg = _edge_segsum(_NRP, epI, "deg", 64)

  deg2 = seg_deg(dstI, zR)

  iP = i0
  iM = i0
  for l in range(_L):
    hW = _tc_linear(r, W_msg[l, :_D, :], b_msg[l].reshape(1, -1), 1024)
    eW = _tc_linear(e16, W_msg[l, _D:, :], jnp.zeros((1, _D), f32), 2048)
    aggR = seg_r2r(hW, srcR, dstR, eW, zR)
    r = _tc_update(aggR, W_upd[l], b_upd[l].reshape(1, -1), r, 1024)
    hPin, hMin = _tc_add2(iP, iM, r, 1024)
    agg2 = seg_i2i(hPin, hMin, srcI, dstI, zR)
    iP, iM = _tc_gcn2(agg2, deg2, hPin, hMin, W_iP[l], b_iP[l].reshape(1, -1),
                      W_iM[l], b_iM[l].reshape(1, -1), 1024)

  dd2 = seg_i2d(iP, iM, srcID, dstID, zD)
  d_P, d_M = dd2[0], dd2[1]
  aggDD2 = seg_d2d(_tc_scale4(d_P, 1024), _tc_scale4(d_M, 1024),
                   srcDD, dstDD, zD)
  h_P, h_M = _tc_update2(aggDD2, W_dP, b_dP.reshape(1, -1),
                         W_dM, b_dM.reshape(1, -1), d_P, d_M, 1024)

  scores = []
  for h_d, W1, b1, W2, b2, Wpl, bpl, g, bet in (
      (h_P, W1P, b1P, W2P, b2P, WplP, bplP, gP, betP),
      (h_M, W1M, b1M, W2M, b2M, WplM, bplM, gM, betM)):
    w2p = jnp.zeros((_D // 2, _D), f32).at[:, :3].set(W2)
    b2p = jnp.zeros((1, _D), f32).at[0, :3].set(b2)
    wplt = jnp.zeros((1, _D), f32).at[0, :3].set(Wpl[:, 0])
    scores.append(_tc_head(h_d, W1, b1.reshape(1, -1), w2p, b2p, wplt,
                           bpl.reshape(1, 1), g.reshape(1, 1),
                           bet.reshape(1, 1)))
  scoreP, scoreM = scores
  ss2 = seg_d2r(jnp.pad(scoreP, ((0, 0), (0, _D - 1))),
                jnp.pad(scoreM, ((0, 0), (0, _D - 1))), srcDR, dstDR, zR)

  rP, rM = _tc_final(ss2, r, 1024)
  return (rP[:_NR], rM[:_NR], scoreP[:_ND], scoreM[:_ND])


# speedup vs baseline: 1.2312x; 1.2312x over previous
import functools

import jax
import jax.numpy as jnp
from jax import lax
from jax.experimental import pallas as pl
from jax.experimental.pallas import tpu as pltpu
from jax.experimental.pallas import tpu_sc as plsc

_NR = 10000
_NI = 10000
_ND = 5000
_D = 128
_DE = 16
_L = 2
_ALPHA = 0.1

_NRP = 10240
_NDP = 5120
_NC = 2
_NS = 16
_NW = _NC * _NS
_CHUNK = 128

_LN_ALPHA = -2.302585092994046


_NBUF = 4
_GS = 8


def _edge_segsum(n_out_pad, e_pad, mode, chunk=_CHUNK, d=_D):
  per_core = mode == "pm"
  et = e_pad // (_NS if per_core else _NW)
  nchunk = et // chunk
  ngrp = nchunk // _GS
  rpt = n_out_pad // _NS

  scratch = [
      pltpu.VMEM((_GS, chunk), jnp.int32),
      pltpu.VMEM((_GS, chunk), jnp.int32),
  ]
  nrows = 1 if mode == "deg" else _NBUF
  scratch += [pltpu.VMEM((chunk, d), jnp.float32) for _ in range(nrows)]
  if mode == "msg":
    scratch += [pltpu.VMEM((chunk, d), jnp.float32) for _ in range(_NBUF)]
  nsem = 2 * _NBUF + (_NBUF if mode == "msg" else 0)
  scratch += [pltpu.SemaphoreType.DMA for _ in range(nsem)]
  scratch.append(pltpu.VMEM_SHARED((n_out_pad, d), jnp.float32))
  mesh = plsc.VectorSubcoreMesh(core_axis_name="c", subcore_axis_name="s")
  out_type = jax.ShapeDtypeStruct((_NC, n_out_pad, d), jnp.float32)

  def body(*refs):
    if mode == "msg":
      table, src2, dst2, ew, zrows, out = refs[:6]
      sc = refs[6:]
      srcb, dstb = sc[0], sc[1]
      rows = sc[2:2 + _NBUF]
      ewb = sc[2 + _NBUF:2 + 2 * _NBUF]
      gsem = sc[2 + 2 * _NBUF:2 + 3 * _NBUF]
      ssem = sc[2 + 3 * _NBUF:2 + 4 * _NBUF]
      esem = sc[2 + 4 * _NBUF:2 + 5 * _NBUF]
      acc = sc[-1]
    elif mode == "pm":
      tab0, tab1, src2, dst2, zrows, out = refs[:6]
      sc = refs[6:]
      srcb, dstb = sc[0], sc[1]
      rows = sc[2:2 + _NBUF]
      gsem = sc[2 + _NBUF:2 + 2 * _NBUF]
      ssem = sc[2 + 2 * _NBUF:2 + 3 * _NBUF]
      acc = sc[-1]
    elif mode == "deg":
      dst2, zrows, out = refs[:3]
      sc = refs[3:]
      srcb, dstb = sc[0], sc[1]
      rows = sc[2:3]
      gsem = sc[3:3 + _NBUF]
      ssem = sc[3 + _NBUF:3 + 2 * _NBUF]
      acc = sc[-1]
    else:
      table, src2, dst2, zrows, out = refs[:5]
      sc = refs[5:]
      srcb, dstb = sc[0], sc[1]
      rows = sc[2:2 + _NBUF]
      gsem = sc[2 + _NBUF:2 + 2 * _NBUF]
      ssem = sc[2 + 2 * _NBUF:2 + 3 * _NBUF]
      acc = sc[-1]
    cid = lax.axis_index("c")
    sid = lax.axis_index("s")
    wid = sid * _NC + cid
    row0 = sid * rpt
    pltpu.sync_copy(zrows.at[pl.ds(row0, rpt)], acc.at[pl.ds(row0, rpt)])
    plsc.subcore_barrier()

    if mode == "deg":
      def init_ones(i, c0):
        rows[0][i, pl.ds(0, 16)] = jnp.full((16,), 1.0, jnp.float32)
        return c0
      lax.fori_loop(0, chunk, init_ones, 0)

    def make_group(table):
      def group(g, carry):
        base = (sid if per_core else wid) * nchunk + g * _GS
        if mode != "deg":
          pltpu.sync_copy(src2.at[pl.ds(base, _GS)], srcb)
        pltpu.sync_copy(dst2.at[pl.ds(base, _GS)], dstb)

        def issue(b, j):
          if mode != "deg":
            pltpu.async_copy(table.at[srcb.at[j]], rows[b], gsem[b])
          if mode == "msg":
            pltpu.async_copy(ew.at[pl.ds((base + j) * chunk, chunk)],
                             ewb[b], esem[b])

        for b in range(_NBUF):
          issue(b, b)
        for half in range(_GS // _NBUF):
          for b in range(_NBUF):
            j = half * _NBUF + b
            if mode != "deg":
              pltpu.make_async_copy(table.at[srcb.at[j]], rows[b],
                                    gsem[b]).wait()
            if mode == "msg":
              pltpu.make_async_copy(ew.at[pl.ds((base + j) * chunk, chunk)],
                                    ewb[b], esem[b]).wait()
              rb, eb = rows[b], ewb[b]

              def ebody(i, c2, rb=rb, eb=eb):
                for kk in range(d // 16):
                  sl = pl.ds(kk * 16, 16)
                  rb[i, sl] = jnp.maximum(rb[i, sl] + eb[i, sl], 0.0)
                return c2

              lax.fori_loop(0, chunk, ebody, 0)
            rsrc = rows[0] if mode == "deg" else rows[b]
            pltpu.async_copy(rsrc, acc.at[dstb.at[j]], ssem[b], add=True)
          for b in range(_NBUF):
            j = half * _NBUF + b
            rsrc = rows[0] if mode == "deg" else rows[b]
            pltpu.make_async_copy(rsrc, acc.at[dstb.at[j]], ssem[b]).wait()
            if half < _GS // _NBUF - 1:
              issue(b, (half + 1) * _NBUF + b)
        return carry
      return group

    if mode == "pm":
      @pl.when(cid == 0)
      def _():
        lax.fori_loop(0, ngrp, make_group(tab0), 0)

      @pl.when(cid == 1)
      def _():
        lax.fori_loop(0, ngrp, make_group(tab1), 0)
    else:
      lax.fori_loop(0, ngrp, make_group(None if mode == "deg" else table), 0)
    plsc.subcore_barrier()
    pltpu.sync_copy(acc.at[pl.ds(row0, rpt)], out.at[cid, pl.ds(row0, rpt)])

  return pl.kernel(body, out_type=out_type, mesh=mesh, scratch_types=scratch)


def _mm(a, b):
  return lax.dot_general(a, b, (((1,), (0,)), ((), ())),
                         preferred_element_type=jnp.float32)


def _tc_linear(x, w, b, bs):
  n, k = x.shape
  m = w.shape[1]

  def body(x_ref, w_ref, b_ref, o_ref):
    o_ref[...] = _mm(x_ref[...], w_ref[...]) + b_ref[...]

  return pl.pallas_call(
      body, grid=(n // bs,),
      in_specs=[pl.BlockSpec((bs, k), lambda i: (i, 0)),
                pl.BlockSpec((k, m), lambda i: (0, 0)),
                pl.BlockSpec((1, m), lambda i: (0, 0))],
      out_specs=pl.BlockSpec((bs, m), lambda i: (i, 0)),
      out_shape=jax.ShapeDtypeStruct((n, m), jnp.float32),
  )(x, w, b)


def _tc_update(agg, w, b, res, bs):
  n = res.shape[0]

  def body(a_ref, w_ref, b_ref, r_ref, o_ref):
    x = a_ref[0] + a_ref[1]
    o_ref[...] = jnp.maximum(_mm(x, w_ref[...]) + b_ref[...], 0.0) + r_ref[...]

  return pl.pallas_call(
      body, grid=(n // bs,),
      in_specs=[pl.BlockSpec((2, bs, _D), lambda i: (0, i, 0)),
                pl.BlockSpec((_D, _D), lambda i: (0, 0)),
                pl.BlockSpec((1, _D), lambda i: (0, 0)),
                pl.BlockSpec((bs, _D), lambda i: (i, 0))],
      out_specs=pl.BlockSpec((bs, _D), lambda i: (i, 0)),
      out_shape=jax.ShapeDtypeStruct((n, _D), jnp.float32),
  )(agg, w, b, res)


def _tc_gcn2(agg2, deg2, hp, hm, wp, bp, wm, bm, bs):
  n = hp.shape[0]

  def body(a_ref, d_ref, hp_ref, hm_ref, wp_ref, bp_ref, wm_ref, bm_ref,
           op_ref, om_ref):
    inv = 1.0 / jnp.maximum(d_ref[0, :, :1] + d_ref[1, :, :1], 1.0)
    xp = a_ref[0] * inv
    xm = a_ref[1] * inv
    op_ref[...] = jnp.maximum(_mm(xp, wp_ref[...]) + bp_ref[...], 0.0) + hp_ref[...]
    om_ref[...] = jnp.maximum(_mm(xm, wm_ref[...]) + bm_ref[...], 0.0) + hm_ref[...]

  s2 = pl.BlockSpec((2, bs, _D), lambda i: (0, i, 0))
  s2d = pl.BlockSpec((2, bs, _D), lambda i: (0, i, 0))
  sd = pl.BlockSpec((bs, _D), lambda i: (i, 0))
  sw = pl.BlockSpec((_D, _D), lambda i: (0, 0))
  sb = pl.BlockSpec((1, _D), lambda i: (0, 0))
  sh = jax.ShapeDtypeStruct((n, _D), jnp.float32)
  return pl.pallas_call(
      body, grid=(n // bs,),
      in_specs=[s2, s2d, sd, sd, sw, sb, sw, sb],
      out_specs=[sd, sd], out_shape=[sh, sh],
  )(agg2, deg2, hp, hm, wp, bp, wm, bm)


def _tc_update2(agg2, wp, bp, wm, bm, rp, rm, bs):
  n = rp.shape[0]

  def body(a_ref, wp_ref, bp_ref, wm_ref, bm_ref, rp_ref, rm_ref,
           op_ref, om_ref):
    op_ref[...] = jnp.maximum(_mm(a_ref[0], wp_ref[...]) + bp_ref[...], 0.0) + rp_ref[...]
    om_ref[...] = jnp.maximum(_mm(a_ref[1], wm_ref[...]) + bm_ref[...], 0.0) + rm_ref[...]

  s2 = pl.BlockSpec((2, bs, _D), lambda i: (0, i, 0))
  sd = pl.BlockSpec((bs, _D), lambda i: (i, 0))
  sw = pl.BlockSpec((_D, _D), lambda i: (0, 0))
  sb = pl.BlockSpec((1, _D), lambda i: (0, 0))
  sh = jax.ShapeDtypeStruct((n, _D), jnp.float32)
  return pl.pallas_call(
      body, grid=(n // bs,),
      in_specs=[s2, sw, sb, sw, sb, sd, sd],
      out_specs=[sd, sd], out_shape=[sh, sh],
  )(agg2, wp, bp, wm, bm, rp, rm)


def _tc_add2(a, b, r, bs):
  n = r.shape[0]

  def body(a_ref, b_ref, r_ref, o1_ref, o2_ref):
    o1_ref[...] = a_ref[...] + r_ref[...]
    o2_ref[...] = b_ref[...] + r_ref[...]

  sp = pl.BlockSpec((bs, _D), lambda i: (i, 0))
  sh = jax.ShapeDtypeStruct((n, _D), jnp.float32)
  return pl.pallas_call(
      body, grid=(n // bs,),
      in_specs=[sp, sp, sp], out_specs=[sp, sp], out_shape=[sh, sh],
  )(a, b, r)


def _tc_scale4(d, bs):
  n = d.shape[0]
  nb = n // bs

  def body(d_ref, o_ref):
    k = (pl.program_id(0) // nb).astype(jnp.float32)
    o_ref[...] = d_ref[...] * jnp.exp(k * _LN_ALPHA)

  return pl.pallas_call(
      body, grid=(4 * nb,),
      in_specs=[pl.BlockSpec((bs, _D), lambda j: (j % nb, 0))],
      out_specs=pl.BlockSpec((bs, _D), lambda j: (j, 0)),
      out_shape=jax.ShapeDtypeStruct((4 * n, _D), jnp.float32),
  )(d)


def _leaky(x):
  return jnp.where(x >= 0.0, x, 0.01 * x)


def _tc_head(x, w1, b1, w2p, b2p, wplt, bpl, g, bet):
  n = x.shape[0]

  def body(x_ref, w1_ref, b1_ref, w2_ref, b2_ref, wp_ref, bp_ref,
           g_ref, be_ref, o_ref):
    s1 = _leaky(_mm(x_ref[...], w1_ref[...]) + b1_ref[...])
    s2 = _leaky(_mm(s1, w2_ref[...]) + b2_ref[...])
    s3 = jnp.sum(s2 * wp_ref[...], axis=1, keepdims=True) + bp_ref[...]
    mask = lax.broadcasted_iota(jnp.int32, (n, 1), 0) < _ND
    cnt = jnp.float32(_ND)
    mean = jnp.sum(jnp.where(mask, s3, 0.0)) / cnt
    dev = jnp.where(mask, s3 - mean, 0.0)
    var = jnp.sum(dev * dev) / cnt
    s = (s3 - mean) * lax.rsqrt(var + 1e-5) * g_ref[...] + be_ref[...]
    o_ref[...] = 1.0 / (1.0 + jnp.exp(-s))

  full = lambda shape: pl.BlockSpec(shape, lambda: tuple(0 for _ in shape))
  return pl.pallas_call(
      body,
      in_specs=[full((n, _D)), full((_D, _D // 2)), full((1, _D // 2)),
                full((_D // 2, _D)), full((1, _D)), full((1, _D)),
                full((1, 1)), full((1, 1)), full((1, 1))],
      out_specs=full((n, 1)),
      out_shape=jax.ShapeDtypeStruct((n, 1), jnp.float32),
  )(x, w1, b1, w2p, b2p, wplt, bpl, g, bet)


def _tc_final(ss2, r, bs):
  n = r.shape[0]

  def body(s_ref, r_ref, o1_ref, o2_ref):
    o1_ref[...] = s_ref[0, :, :1] * r_ref[...]
    o2_ref[...] = s_ref[1, :, :1] * r_ref[...]

  s32 = pl.BlockSpec((2, bs, _D), lambda i: (0, i, 0))
  sd = pl.BlockSpec((bs, _D), lambda i: (i, 0))
  sh = jax.ShapeDtypeStruct((n, _D), jnp.float32)
  return pl.pallas_call(
      body, grid=(n // bs,),
      in_specs=[s32, sd], out_specs=[sd, sd], out_shape=[sh, sh],
  )(ss2, r)


def _pad_ei(ei, dummy, chunk):
  e = ei.shape[1]
  ep = -(-e // (_NW * chunk * _GS)) * (_NW * chunk * _GS)
  src = jnp.pad(ei[0].astype(jnp.int32), (0, ep - e))
  dst = jnp.pad(ei[1].astype(jnp.int32), (0, ep - e), constant_values=dummy)
  return src.reshape(ep // chunk, chunk), dst.reshape(ep // chunk, chunk), ep


def kernel(r_node, r2r_edge, i_node, d2d_edge, r2r_ei, i2i_ei, d2d_ei, i2d_ei,
           d2r_ei, W_msg, b_msg, W_upd, b_upd, W_iP, b_iP, W_iM, b_iM, W_dP,
           b_dP, W_dM, b_dM, W1P, b1P, W2P, b2P, WplP, bplP, W1M, b1M, W2M,
           b2M, WplM, bplM, gP, betP, gM, betM):
  f32 = jnp.float32

  r = jnp.pad(r_node, ((0, _NRP - _NR), (0, 0)))
  i0 = jnp.pad(i_node, ((0, _NRP - _NI), (0, 0)))

  srcR, dstR, epR = _pad_ei(r2r_ei, _NR, 32)
  srcI, dstI, epI = _pad_ei(i2i_ei, _NI, 64)
  srcID, dstID, epID = _pad_ei(i2d_ei, _ND, 128)
  srcDD, dstDD, epDD = _pad_ei(d2d_ei, _ND, 128)
  srcDR, dstDR, epDR = _pad_ei(d2r_ei, _NR, 64)
  distp = jnp.pad(d2d_edge.astype(jnp.int32), (0, epDD - d2d_edge.shape[0]))
  srcDD = (srcDD.reshape(-1) + distp * _NDP).reshape(-1, 128)
  e16 = jnp.pad(r2r_edge, ((0, epR - r2r_edge.shape[0]), (0, 0)))

  zR = jnp.zeros((_NRP, _D), f32)
  zD = jnp.zeros((_NDP, _D), f32)

  seg_r2r = _edge_segsum(_NRP, epR, "msg", 32)
  seg_i2i = _edge_segsum(_NRP, epI, "pm", 64)
  seg_i2d = _edge_segsum(_NDP, epID, "pm", 128)
  seg_d2d = _edge_segsum(_NDP, epDD, "pm", 128)
  seg_d2r = _edge_segsum(_NRP, epDR, "pm", 64)
  seg_deg = _edge_segsum(_NRP, epI, "deg", 64)

  deg2 = seg_deg(dstI, zR)

  iP = i0
  iM = i0
  for l in range(_L):
    hW = _tc_linear(r, W_msg[l, :_D, :], b_msg[l].reshape(1, -1), 1024)
    eW = _tc_linear(e16, W_msg[l, _D:, :], jnp.zeros((1, _D), f32), 2048)
    aggR = seg_r2r(hW, srcR, dstR, eW, zR)
    r = _tc_update(aggR, W_upd[l], b_upd[l].reshape(1, -1), r, 1024)
    hPin, hMin = _tc_add2(iP, iM, r, 1024)
    agg2 = seg_i2i(hPin, hMin, srcI, dstI, zR)
    iP, iM = _tc_gcn2(agg2, deg2, hPin, hMin, W_iP[l], b_iP[l].reshape(1, -1),
                      W_iM[l], b_iM[l].reshape(1, -1), 1024)

  dd2 = seg_i2d(iP, iM, srcID, dstID, zD)
  d_P, d_M = dd2[0], dd2[1]
  aggDD2 = seg_d2d(_tc_scale4(d_P, 1024), _tc_scale4(d_M, 1024),
                   srcDD, dstDD, zD)
  h_P, h_M = _tc_update2(aggDD2, W_dP, b_dP.reshape(1, -1),
                         W_dM, b_dM.reshape(1, -1), d_P, d_M, 1024)

  scores = []
  for h_d, W1, b1, W2, b2, Wpl, bpl, g, bet in (
      (h_P, W1P, b1P, W2P, b2P, WplP, bplP, gP, betP),
      (h_M, W1M, b1M, W2M, b2M, WplM, bplM, gM, betM)):
    w2p = jnp.zeros((_D // 2, _D), f32).at[:, :3].set(W2)
    b2p = jnp.zeros((1, _D), f32).at[0, :3].set(b2)
    wplt = jnp.zeros((1, _D), f32).at[0, :3].set(Wpl[:, 0])
    scores.append(_tc_head(h_d, W1, b1.reshape(1, -1), w2p, b2p, wplt,
                           bpl.reshape(1, 1), g.reshape(1, 1),
                           bet.reshape(1, 1)))
  scoreP, scoreM = scores
  ss2 = seg_d2r(jnp.pad(scoreP, ((0, 0), (0, _D - 1))),
                jnp.pad(scoreM, ((0, 0), (0, _D - 1))), srcDR, dstDR, zR)

  rP, rM = _tc_final(ss2, r, 1024)
  return (rP[:_NR], rM[:_NR], scoreP[:_ND], scoreM[:_ND])


# speedup vs baseline: 1.2797x; 1.0393x over previous
import functools

import jax
import jax.numpy as jnp
from jax import lax
from jax.experimental import pallas as pl
from jax.experimental.pallas import tpu as pltpu
from jax.experimental.pallas import tpu_sc as plsc

_NR = 10000
_NI = 10000
_ND = 5000
_D = 128
_DE = 16
_L = 2
_ALPHA = 0.1

_NRP = 10240
_NDP = 5120
_NC = 2
_NS = 16
_NW = _NC * _NS
_CHUNK = 128

_LN_ALPHA = -2.302585092994046


_NBUF = 4
_GS = 8


def _edge_segsum(n_out_pad, e_pad, mode, chunk=_CHUNK, d=_D, nbuf=_NBUF):
  per_core = mode == "pm"
  et = e_pad // (_NS if per_core else _NW)
  nchunk = et // chunk
  ngrp = nchunk // _GS
  rpt = n_out_pad // _NS

  scratch = [
      pltpu.VMEM((_GS, chunk), jnp.int32),
      pltpu.VMEM((_GS, chunk), jnp.int32),
  ]
  nrows = 1 if mode == "deg" else nbuf
  scratch += [pltpu.VMEM((chunk, d), jnp.float32) for _ in range(nrows)]
  if mode == "msg":
    scratch += [pltpu.VMEM((chunk, d), jnp.float32) for _ in range(nbuf)]
  nsem = 2 * nbuf + (nbuf if mode == "msg" else 0)
  scratch += [pltpu.SemaphoreType.DMA for _ in range(nsem)]
  scratch.append(pltpu.VMEM_SHARED((n_out_pad, d), jnp.float32))
  mesh = plsc.VectorSubcoreMesh(core_axis_name="c", subcore_axis_name="s")
  out_type = jax.ShapeDtypeStruct((_NC, n_out_pad, d), jnp.float32)

  def body(*refs):
    if mode == "msg":
      table, src2, dst2, ew, zrows, out = refs[:6]
      sc = refs[6:]
      srcb, dstb = sc[0], sc[1]
      rows = sc[2:2 + nbuf]
      ewb = sc[2 + nbuf:2 + 2 * nbuf]
      gsem = sc[2 + 2 * nbuf:2 + 3 * nbuf]
      ssem = sc[2 + 3 * nbuf:2 + 4 * nbuf]
      esem = sc[2 + 4 * nbuf:2 + 5 * nbuf]
      acc = sc[-1]
    elif mode == "pm":
      tab0, tab1, src2, dst2, zrows, out = refs[:6]
      sc = refs[6:]
      srcb, dstb = sc[0], sc[1]
      rows = sc[2:2 + nbuf]
      gsem = sc[2 + nbuf:2 + 2 * nbuf]
      ssem = sc[2 + 2 * nbuf:2 + 3 * nbuf]
      acc = sc[-1]
    elif mode == "deg":
      dst2, zrows, out = refs[:3]
      sc = refs[3:]
      srcb, dstb = sc[0], sc[1]
      rows = sc[2:3]
      gsem = sc[3:3 + nbuf]
      ssem = sc[3 + nbuf:3 + 2 * nbuf]
      acc = sc[-1]
    else:
      table, src2, dst2, zrows, out = refs[:5]
      sc = refs[5:]
      srcb, dstb = sc[0], sc[1]
      rows = sc[2:2 + nbuf]
      gsem = sc[2 + nbuf:2 + 2 * nbuf]
      ssem = sc[2 + 2 * nbuf:2 + 3 * nbuf]
      acc = sc[-1]
    cid = lax.axis_index("c")
    sid = lax.axis_index("s")
    wid = sid * _NC + cid
    row0 = sid * rpt
    pltpu.sync_copy(zrows.at[pl.ds(row0, rpt)], acc.at[pl.ds(row0, rpt)])
    plsc.subcore_barrier()

    if mode == "deg":
      def init_ones(i, c0):
        rows[0][i, pl.ds(0, 16)] = jnp.full((16,), 1.0, jnp.float32)
        return c0
      lax.fori_loop(0, chunk, init_ones, 0)

    def make_group(table):
      def group(g, carry):
        base = (sid if per_core else wid) * nchunk + g * _GS
        if mode != "deg":
          pltpu.sync_copy(src2.at[pl.ds(base, _GS)], srcb)
        pltpu.sync_copy(dst2.at[pl.ds(base, _GS)], dstb)

        def issue(b, j):
          if mode != "deg":
            pltpu.async_copy(table.at[srcb.at[j]], rows[b], gsem[b])
          if mode == "msg":
            pltpu.async_copy(ew.at[pl.ds((base + j) * chunk, chunk)],
                             ewb[b], esem[b])

        for b in range(nbuf):
          issue(b, b)
        for half in range(_GS // nbuf):
          for b in range(nbuf):
            j = half * nbuf + b
            if mode != "deg":
              pltpu.make_async_copy(table.at[srcb.at[j]], rows[b],
                                    gsem[b]).wait()
            if mode == "msg":
              pltpu.make_async_copy(ew.at[pl.ds((base + j) * chunk, chunk)],
                                    ewb[b], esem[b]).wait()
              rb, eb = rows[b], ewb[b]

              def ebody(i, c2, rb=rb, eb=eb):
                for kk in range(d // 16):
                  sl = pl.ds(kk * 16, 16)
                  rb[i, sl] = jnp.maximum(rb[i, sl] + eb[i, sl], 0.0)
                return c2

              lax.fori_loop(0, chunk, ebody, 0)
            rsrc = rows[0] if mode == "deg" else rows[b]
            pltpu.async_copy(rsrc, acc.at[dstb.at[j]], ssem[b], add=True)
          for b in range(nbuf):
            j = half * nbuf + b
            rsrc = rows[0] if mode == "deg" else rows[b]
            pltpu.make_async_copy(rsrc, acc.at[dstb.at[j]], ssem[b]).wait()
            if half < _GS // nbuf - 1:
              issue(b, (half + 1) * nbuf + b)
        return carry
      return group

    if mode == "pm":
      @pl.when(cid == 0)
      def _():
        lax.fori_loop(0, ngrp, make_group(tab0), 0)

      @pl.when(cid == 1)
      def _():
        lax.fori_loop(0, ngrp, make_group(tab1), 0)
    else:
      lax.fori_loop(0, ngrp, make_group(None if mode == "deg" else table), 0)
    plsc.subcore_barrier()
    pltpu.sync_copy(acc.at[pl.ds(row0, rpt)], out.at[cid, pl.ds(row0, rpt)])

  return pl.kernel(body, out_type=out_type, mesh=mesh, scratch_types=scratch)


def _mm(a, b):
  return lax.dot_general(a, b, (((1,), (0,)), ((), ())),
                         preferred_element_type=jnp.float32)


def _tc_linear(x, w, b, bs):
  n, k = x.shape
  m = w.shape[1]

  def body(x_ref, w_ref, b_ref, o_ref):
    o_ref[...] = _mm(x_ref[...], w_ref[...]) + b_ref[...]

  return pl.pallas_call(
      body, grid=(n // bs,),
      in_specs=[pl.BlockSpec((bs, k), lambda i: (i, 0)),
                pl.BlockSpec((k, m), lambda i: (0, 0)),
                pl.BlockSpec((1, m), lambda i: (0, 0))],
      out_specs=pl.BlockSpec((bs, m), lambda i: (i, 0)),
      out_shape=jax.ShapeDtypeStruct((n, m), jnp.float32),
  )(x, w, b)


def _tc_update(agg, w, b, res, bs):
  n = res.shape[0]

  def body(a_ref, w_ref, b_ref, r_ref, o_ref):
    x = a_ref[0] + a_ref[1]
    o_ref[...] = jnp.maximum(_mm(x, w_ref[...]) + b_ref[...], 0.0) + r_ref[...]

  return pl.pallas_call(
      body, grid=(n // bs,),
      in_specs=[pl.BlockSpec((2, bs, _D), lambda i: (0, i, 0)),
                pl.BlockSpec((_D, _D), lambda i: (0, 0)),
                pl.BlockSpec((1, _D), lambda i: (0, 0)),
                pl.BlockSpec((bs, _D), lambda i: (i, 0))],
      out_specs=pl.BlockSpec((bs, _D), lambda i: (i, 0)),
      out_shape=jax.ShapeDtypeStruct((n, _D), jnp.float32),
  )(agg, w, b, res)


def _tc_gcn2(agg2, deg2, hp, hm, wp, bp, wm, bm, bs):
  n = hp.shape[0]

  def body(a_ref, d_ref, hp_ref, hm_ref, wp_ref, bp_ref, wm_ref, bm_ref,
           op_ref, om_ref):
    inv = 1.0 / jnp.maximum(d_ref[0, :, :1] + d_ref[1, :, :1], 1.0)
    xp = a_ref[0] * inv
    xm = a_ref[1] * inv
    op_ref[...] = jnp.maximum(_mm(xp, wp_ref[...]) + bp_ref[...], 0.0) + hp_ref[...]
    om_ref[...] = jnp.maximum(_mm(xm, wm_ref[...]) + bm_ref[...], 0.0) + hm_ref[...]

  s2 = pl.BlockSpec((2, bs, _D), lambda i: (0, i, 0))
  s2d = pl.BlockSpec((2, bs, _D), lambda i: (0, i, 0))
  sd = pl.BlockSpec((bs, _D), lambda i: (i, 0))
  sw = pl.BlockSpec((_D, _D), lambda i: (0, 0))
  sb = pl.BlockSpec((1, _D), lambda i: (0, 0))
  sh = jax.ShapeDtypeStruct((n, _D), jnp.float32)
  return pl.pallas_call(
      body, grid=(n // bs,),
      in_specs=[s2, s2d, sd, sd, sw, sb, sw, sb],
      out_specs=[sd, sd], out_shape=[sh, sh],
  )(agg2, deg2, hp, hm, wp, bp, wm, bm)


def _tc_update2(agg2, wp, bp, wm, bm, rp, rm, bs):
  n = rp.shape[0]

  def body(a_ref, wp_ref, bp_ref, wm_ref, bm_ref, rp_ref, rm_ref,
           op_ref, om_ref):
    op_ref[...] = jnp.maximum(_mm(a_ref[0], wp_ref[...]) + bp_ref[...], 0.0) + rp_ref[...]
    om_ref[...] = jnp.maximum(_mm(a_ref[1], wm_ref[...]) + bm_ref[...], 0.0) + rm_ref[...]

  s2 = pl.BlockSpec((2, bs, _D), lambda i: (0, i, 0))
  sd = pl.BlockSpec((bs, _D), lambda i: (i, 0))
  sw = pl.BlockSpec((_D, _D), lambda i: (0, 0))
  sb = pl.BlockSpec((1, _D), lambda i: (0, 0))
  sh = jax.ShapeDtypeStruct((n, _D), jnp.float32)
  return pl.pallas_call(
      body, grid=(n // bs,),
      in_specs=[s2, sw, sb, sw, sb, sd, sd],
      out_specs=[sd, sd], out_shape=[sh, sh],
  )(agg2, wp, bp, wm, bm, rp, rm)


def _tc_add2(a, b, r, bs):
  n = r.shape[0]

  def body(a_ref, b_ref, r_ref, o1_ref, o2_ref):
    o1_ref[...] = a_ref[...] + r_ref[...]
    o2_ref[...] = b_ref[...] + r_ref[...]

  sp = pl.BlockSpec((bs, _D), lambda i: (i, 0))
  sh = jax.ShapeDtypeStruct((n, _D), jnp.float32)
  return pl.pallas_call(
      body, grid=(n // bs,),
      in_specs=[sp, sp, sp], out_specs=[sp, sp], out_shape=[sh, sh],
  )(a, b, r)


def _tc_scale4(d, bs):
  n = d.shape[0]
  nb = n // bs

  def body(d_ref, o_ref):
    k = (pl.program_id(0) // nb).astype(jnp.float32)
    o_ref[...] = d_ref[...] * jnp.exp(k * _LN_ALPHA)

  return pl.pallas_call(
      body, grid=(4 * nb,),
      in_specs=[pl.BlockSpec((bs, _D), lambda j: (j % nb, 0))],
      out_specs=pl.BlockSpec((bs, _D), lambda j: (j, 0)),
      out_shape=jax.ShapeDtypeStruct((4 * n, _D), jnp.float32),
  )(d)


def _leaky(x):
  return jnp.where(x >= 0.0, x, 0.01 * x)


def _tc_head(x, w1, b1, w2p, b2p, wplt, bpl, g, bet):
  n = x.shape[0]

  def body(x_ref, w1_ref, b1_ref, w2_ref, b2_ref, wp_ref, bp_ref,
           g_ref, be_ref, o_ref):
    s1 = _leaky(_mm(x_ref[...], w1_ref[...]) + b1_ref[...])
    s2 = _leaky(_mm(s1, w2_ref[...]) + b2_ref[...])
    s3 = jnp.sum(s2 * wp_ref[...], axis=1, keepdims=True) + bp_ref[...]
    mask = lax.broadcasted_iota(jnp.int32, (n, 1), 0) < _ND
    cnt = jnp.float32(_ND)
    mean = jnp.sum(jnp.where(mask, s3, 0.0)) / cnt
    dev = jnp.where(mask, s3 - mean, 0.0)
    var = jnp.sum(dev * dev) / cnt
    s = (s3 - mean) * lax.rsqrt(var + 1e-5) * g_ref[...] + be_ref[...]
    o_ref[...] = 1.0 / (1.0 + jnp.exp(-s))

  full = lambda shape: pl.BlockSpec(shape, lambda: tuple(0 for _ in shape))
  return pl.pallas_call(
      body,
      in_specs=[full((n, _D)), full((_D, _D // 2)), full((1, _D // 2)),
                full((_D // 2, _D)), full((1, _D)), full((1, _D)),
                full((1, 1)), full((1, 1)), full((1, 1))],
      out_specs=full((n, 1)),
      out_shape=jax.ShapeDtypeStruct((n, 1), jnp.float32),
  )(x, w1, b1, w2p, b2p, wplt, bpl, g, bet)


def _tc_final(ss2, r, bs):
  n = r.shape[0]

  def body(s_ref, r_ref, o1_ref, o2_ref):
    o1_ref[...] = s_ref[0, :, :1] * r_ref[...]
    o2_ref[...] = s_ref[1, :, :1] * r_ref[...]

  s32 = pl.BlockSpec((2, bs, _D), lambda i: (0, i, 0))
  sd = pl.BlockSpec((bs, _D), lambda i: (i, 0))
  sh = jax.ShapeDtypeStruct((n, _D), jnp.float32)
  return pl.pallas_call(
      body, grid=(n // bs,),
      in_specs=[s32, sd], out_specs=[sd, sd], out_shape=[sh, sh],
  )(ss2, r)


def _pad_ei(ei, dummy, chunk):
  e = ei.shape[1]
  ep = -(-e // (_NW * chunk * _GS)) * (_NW * chunk * _GS)
  src = jnp.pad(ei[0].astype(jnp.int32), (0, ep - e))
  dst = jnp.pad(ei[1].astype(jnp.int32), (0, ep - e), constant_values=dummy)
  return src.reshape(ep // chunk, chunk), dst.reshape(ep // chunk, chunk), ep


def kernel(r_node, r2r_edge, i_node, d2d_edge, r2r_ei, i2i_ei, d2d_ei, i2d_ei,
           d2r_ei, W_msg, b_msg, W_upd, b_upd, W_iP, b_iP, W_iM, b_iM, W_dP,
           b_dP, W_dM, b_dM, W1P, b1P, W2P, b2P, WplP, bplP, W1M, b1M, W2M,
           b2M, WplM, bplM, gP, betP, gM, betM):
  f32 = jnp.float32

  r = jnp.pad(r_node, ((0, _NRP - _NR), (0, 0)))
  i0 = jnp.pad(i_node, ((0, _NRP - _NI), (0, 0)))

  srcR, dstR, epR = _pad_ei(r2r_ei, _NR, 32)
  srcI, dstI, epI = _pad_ei(i2i_ei, _NI, 128)
  srcID, dstID, epID = _pad_ei(i2d_ei, _ND, 128)
  srcDD, dstDD, epDD = _pad_ei(d2d_ei, _ND, 128)
  srcDR, dstDR, epDR = _pad_ei(d2r_ei, _NR, 128)
  distp = jnp.pad(d2d_edge.astype(jnp.int32), (0, epDD - d2d_edge.shape[0]))
  srcDD = (srcDD.reshape(-1) + distp * _NDP).reshape(-1, 128)
  e16 = jnp.pad(r2r_edge, ((0, epR - r2r_edge.shape[0]), (0, 0)))

  zR = jnp.zeros((_NRP, _D), f32)
  zD = jnp.zeros((_NDP, _D), f32)

  seg_r2r = _edge_segsum(_NRP, epR, "msg", 32)
  seg_i2i = _edge_segsum(_NRP, epI, "pm", 128, nbuf=2)
  seg_i2d = _edge_segsum(_NDP, epID, "pm", 128)
  seg_d2d = _edge_segsum(_NDP, epDD, "pm", 128)
  seg_d2r = _edge_segsum(_NRP, epDR, "pm", 128, nbuf=2)
  seg_deg = _edge_segsum(_NRP, epI, "deg", 128)

  deg2 = seg_deg(dstI, zR)

  iP = i0
  iM = i0
  for l in range(_L):
    hW = _tc_linear(r, W_msg[l, :_D, :], b_msg[l].reshape(1, -1), 1024)
    eW = _tc_linear(e16, W_msg[l, _D:, :], jnp.zeros((1, _D), f32), 2048)
    aggR = seg_r2r(hW, srcR, dstR, eW, zR)
    r = _tc_update(aggR, W_upd[l], b_upd[l].reshape(1, -1), r, 1024)
    hPin, hMin = _tc_add2(iP, iM, r, 1024)
    agg2 = seg_i2i(hPin, hMin, srcI, dstI, zR)
    iP, iM = _tc_gcn2(agg2, deg2, hPin, hMin, W_iP[l], b_iP[l].reshape(1, -1),
                      W_iM[l], b_iM[l].reshape(1, -1), 1024)

  dd2 = seg_i2d(iP, iM, srcID, dstID, zD)
  d_P, d_M = dd2[0], dd2[1]
  aggDD2 = seg_d2d(_tc_scale4(d_P, 1024), _tc_scale4(d_M, 1024),
                   srcDD, dstDD, zD)
  h_P, h_M = _tc_update2(aggDD2, W_dP, b_dP.reshape(1, -1),
                         W_dM, b_dM.reshape(1, -1), d_P, d_M, 1024)

  scores = []
  for h_d, W1, b1, W2, b2, Wpl, bpl, g, bet in (
      (h_P, W1P, b1P, W2P, b2P, WplP, bplP, gP, betP),
      (h_M, W1M, b1M, W2M, b2M, WplM, bplM, gM, betM)):
    w2p = jnp.zeros((_D // 2, _D), f32).at[:, :3].set(W2)
    b2p = jnp.zeros((1, _D), f32).at[0, :3].set(b2)
    wplt = jnp.zeros((1, _D), f32).at[0, :3].set(Wpl[:, 0])
    scores.append(_tc_head(h_d, W1, b1.reshape(1, -1), w2p, b2p, wplt,
                           bpl.reshape(1, 1), g.reshape(1, 1),
                           bet.reshape(1, 1)))
  scoreP, scoreM = scores
  ss2 = seg_d2r(jnp.pad(scoreP, ((0, 0), (0, _D - 1))),
                jnp.pad(scoreM, ((0, 0), (0, _D - 1))), srcDR, dstDR, zR)

  rP, rM = _tc_final(ss2, r, 1024)
  return (rP[:_NR], rM[:_NR], scoreP[:_ND], scoreM[:_ND])


# speedup vs baseline: 1.3644x; 1.0662x over previous
import functools

import jax
import jax.numpy as jnp
from jax import lax
from jax.experimental import pallas as pl
from jax.experimental.pallas import tpu as pltpu
from jax.experimental.pallas import tpu_sc as plsc

_NR = 10000
_NI = 10000
_ND = 5000
_D = 128
_DE = 16
_L = 2
_ALPHA = 0.1

_NRP = 10240
_NDP = 5120
_NC = 2
_NS = 16
_NW = _NC * _NS
_CHUNK = 128

_LN_ALPHA = -2.302585092994046


_NBUF = 4
_GS = 8


def _edge_segsum(n_out_pad, e_pad, mode, chunk=_CHUNK, d=_D, nbuf=_NBUF):
  per_core = mode == "pm"
  et = e_pad // (_NS if per_core else _NW)
  nchunk = et // chunk
  ngrp = nchunk // _GS
  rpt = n_out_pad // _NS

  scratch = [
      pltpu.VMEM((_GS, chunk), jnp.int32),
      pltpu.VMEM((_GS, chunk), jnp.int32),
  ]
  nrows = 1 if mode == "deg" else nbuf
  scratch += [pltpu.VMEM((chunk, d), jnp.float32) for _ in range(nrows)]
  if mode == "msg":
    scratch += [pltpu.VMEM((chunk, d), jnp.float32) for _ in range(nbuf)]
  nsem = 2 * nbuf + (nbuf if mode == "msg" else 0)
  scratch += [pltpu.SemaphoreType.DMA for _ in range(nsem)]
  scratch.append(pltpu.VMEM_SHARED((n_out_pad, d), jnp.float32))
  mesh = plsc.VectorSubcoreMesh(core_axis_name="c", subcore_axis_name="s")
  out_type = jax.ShapeDtypeStruct((_NC, n_out_pad, d), jnp.float32)

  def body(*refs):
    if mode == "msg":
      table, src2, dst2, ew, zrows, out = refs[:6]
      sc = refs[6:]
      srcb, dstb = sc[0], sc[1]
      rows = sc[2:2 + nbuf]
      ewb = sc[2 + nbuf:2 + 2 * nbuf]
      gsem = sc[2 + 2 * nbuf:2 + 3 * nbuf]
      ssem = sc[2 + 3 * nbuf:2 + 4 * nbuf]
      esem = sc[2 + 4 * nbuf:2 + 5 * nbuf]
      acc = sc[-1]
    elif mode == "pm":
      tab0, tab1, src2, dst2, zrows, out = refs[:6]
      sc = refs[6:]
      srcb, dstb = sc[0], sc[1]
      rows = sc[2:2 + nbuf]
      gsem = sc[2 + nbuf:2 + 2 * nbuf]
      ssem = sc[2 + 2 * nbuf:2 + 3 * nbuf]
      acc = sc[-1]
    elif mode == "deg":
      dst2, zrows, out = refs[:3]
      sc = refs[3:]
      srcb, dstb = sc[0], sc[1]
      rows = sc[2:3]
      gsem = sc[3:3 + nbuf]
      ssem = sc[3 + nbuf:3 + 2 * nbuf]
      acc = sc[-1]
    else:
      table, src2, dst2, zrows, out = refs[:5]
      sc = refs[5:]
      srcb, dstb = sc[0], sc[1]
      rows = sc[2:2 + nbuf]
      gsem = sc[2 + nbuf:2 + 2 * nbuf]
      ssem = sc[2 + 2 * nbuf:2 + 3 * nbuf]
      acc = sc[-1]
    cid = lax.axis_index("c")
    sid = lax.axis_index("s")
    wid = sid * _NC + cid
    row0 = sid * rpt
    pltpu.sync_copy(zrows.at[pl.ds(row0, rpt)], acc.at[pl.ds(row0, rpt)])
    plsc.subcore_barrier()

    if mode == "deg":
      def init_ones(i, c0):
        rows[0][i, pl.ds(0, 16)] = jnp.full((16,), 1.0, jnp.float32)
        return c0
      lax.fori_loop(0, chunk, init_ones, 0)

    def make_group(table):
      def group(g, carry):
        base = (sid if per_core else wid) * nchunk + g * _GS
        if mode != "deg":
          pltpu.sync_copy(src2.at[pl.ds(base, _GS)], srcb)
        pltpu.sync_copy(dst2.at[pl.ds(base, _GS)], dstb)

        def issue(b, j):
          if mode != "deg":
            pltpu.async_copy(table.at[srcb.at[j]], rows[b], gsem[b])
          if mode == "msg":
            pltpu.async_copy(ew.at[pl.ds((base + j) * chunk, chunk)],
                             ewb[b], esem[b])

        for b in range(nbuf):
          issue(b, b)
        for half in range(_GS // nbuf):
          for b in range(nbuf):
            j = half * nbuf + b
            if mode != "deg":
              pltpu.make_async_copy(table.at[srcb.at[j]], rows[b],
                                    gsem[b]).wait()
            if mode == "msg":
              pltpu.make_async_copy(ew.at[pl.ds((base + j) * chunk, chunk)],
                                    ewb[b], esem[b]).wait()
              rb, eb = rows[b], ewb[b]

              def ebody(i, c2, rb=rb, eb=eb):
                for kk in range(d // 16):
                  sl = pl.ds(kk * 16, 16)
                  rb[i, sl] = jnp.maximum(rb[i, sl] + eb[i, sl], 0.0)
                return c2

              lax.fori_loop(0, chunk, ebody, 0)
            rsrc = rows[0] if mode == "deg" else rows[b]
            pltpu.async_copy(rsrc, acc.at[dstb.at[j]], ssem[b], add=True)
          for b in range(nbuf):
            j = half * nbuf + b
            rsrc = rows[0] if mode == "deg" else rows[b]
            pltpu.make_async_copy(rsrc, acc.at[dstb.at[j]], ssem[b]).wait()
            if half < _GS // nbuf - 1:
              issue(b, (half + 1) * nbuf + b)
        return carry
      return group

    if mode == "pm":
      @pl.when(cid == 0)
      def _():
        lax.fori_loop(0, ngrp, make_group(tab0), 0)

      @pl.when(cid == 1)
      def _():
        lax.fori_loop(0, ngrp, make_group(tab1), 0)
    else:
      lax.fori_loop(0, ngrp, make_group(None if mode == "deg" else table), 0)
    plsc.subcore_barrier()
    pltpu.sync_copy(acc.at[pl.ds(row0, rpt)], out.at[cid, pl.ds(row0, rpt)])

  return pl.kernel(body, out_type=out_type, mesh=mesh, scratch_types=scratch)


def _mm(a, b):
  return lax.dot_general(a, b, (((1,), (0,)), ((), ())),
                         preferred_element_type=jnp.float32)


def _tc_linear(x, w, b, bs):
  n, k = x.shape
  m = w.shape[1]

  def body(x_ref, w_ref, b_ref, o_ref):
    o_ref[...] = _mm(x_ref[...], w_ref[...]) + b_ref[...]

  return pl.pallas_call(
      body, grid=(n // bs,),
      in_specs=[pl.BlockSpec((bs, k), lambda i: (i, 0)),
                pl.BlockSpec((k, m), lambda i: (0, 0)),
                pl.BlockSpec((1, m), lambda i: (0, 0))],
      out_specs=pl.BlockSpec((bs, m), lambda i: (i, 0)),
      out_shape=jax.ShapeDtypeStruct((n, m), jnp.float32),
  )(x, w, b)


def _tc_update(agg, w, b, res, bs):
  n = res.shape[0]

  def body(a_ref, w_ref, b_ref, r_ref, o_ref):
    x = a_ref[0] + a_ref[1]
    o_ref[...] = jnp.maximum(_mm(x, w_ref[...]) + b_ref[...], 0.0) + r_ref[...]

  return pl.pallas_call(
      body, grid=(n // bs,),
      in_specs=[pl.BlockSpec((2, bs, _D), lambda i: (0, i, 0)),
                pl.BlockSpec((_D, _D), lambda i: (0, 0)),
                pl.BlockSpec((1, _D), lambda i: (0, 0)),
                pl.BlockSpec((bs, _D), lambda i: (i, 0))],
      out_specs=pl.BlockSpec((bs, _D), lambda i: (i, 0)),
      out_shape=jax.ShapeDtypeStruct((n, _D), jnp.float32),
  )(agg, w, b, res)


def _tc_gcn2(agg2, deg2, hp, hm, wp, bp, wm, bm, bs):
  n = hp.shape[0]

  def body(a_ref, d_ref, hp_ref, hm_ref, wp_ref, bp_ref, wm_ref, bm_ref,
           op_ref, om_ref):
    inv = 1.0 / jnp.maximum(d_ref[0, :, :1] + d_ref[1, :, :1], 1.0)
    xp = a_ref[0] * inv
    xm = a_ref[1] * inv
    op_ref[...] = jnp.maximum(_mm(xp, wp_ref[...]) + bp_ref[...], 0.0) + hp_ref[...]
    om_ref[...] = jnp.maximum(_mm(xm, wm_ref[...]) + bm_ref[...], 0.0) + hm_ref[...]

  s2 = pl.BlockSpec((2, bs, _D), lambda i: (0, i, 0))
  s2d = pl.BlockSpec((2, bs, _D), lambda i: (0, i, 0))
  sd = pl.BlockSpec((bs, _D), lambda i: (i, 0))
  sw = pl.BlockSpec((_D, _D), lambda i: (0, 0))
  sb = pl.BlockSpec((1, _D), lambda i: (0, 0))
  sh = jax.ShapeDtypeStruct((n, _D), jnp.float32)
  return pl.pallas_call(
      body, grid=(n // bs,),
      in_specs=[s2, s2d, sd, sd, sw, sb, sw, sb],
      out_specs=[sd, sd], out_shape=[sh, sh],
  )(agg2, deg2, hp, hm, wp, bp, wm, bm)


def _tc_update2(agg2, wp, bp, wm, bm, rp, rm, bs):
  n = rp.shape[0]

  def body(a_ref, wp_ref, bp_ref, wm_ref, bm_ref, rp_ref, rm_ref,
           op_ref, om_ref):
    op_ref[...] = jnp.maximum(_mm(a_ref[0], wp_ref[...]) + bp_ref[...], 0.0) + rp_ref[...]
    om_ref[...] = jnp.maximum(_mm(a_ref[1], wm_ref[...]) + bm_ref[...], 0.0) + rm_ref[...]

  s2 = pl.BlockSpec((2, bs, _D), lambda i: (0, i, 0))
  sd = pl.BlockSpec((bs, _D), lambda i: (i, 0))
  sw = pl.BlockSpec((_D, _D), lambda i: (0, 0))
  sb = pl.BlockSpec((1, _D), lambda i: (0, 0))
  sh = jax.ShapeDtypeStruct((n, _D), jnp.float32)
  return pl.pallas_call(
      body, grid=(n // bs,),
      in_specs=[s2, sw, sb, sw, sb, sd, sd],
      out_specs=[sd, sd], out_shape=[sh, sh],
  )(agg2, wp, bp, wm, bm, rp, rm)


def _tc_add2(a, b, r, bs):
  n = r.shape[0]

  def body(a_ref, b_ref, r_ref, o1_ref, o2_ref):
    o1_ref[...] = a_ref[...] + r_ref[...]
    o2_ref[...] = b_ref[...] + r_ref[...]

  sp = pl.BlockSpec((bs, _D), lambda i: (i, 0))
  sh = jax.ShapeDtypeStruct((n, _D), jnp.float32)
  return pl.pallas_call(
      body, grid=(n // bs,),
      in_specs=[sp, sp, sp], out_specs=[sp, sp], out_shape=[sh, sh],
  )(a, b, r)


def _tc_scale4(d, bs):
  n = d.shape[0]
  nb = n // bs

  def body(d_ref, o_ref):
    k = (pl.program_id(0) // nb).astype(jnp.float32)
    o_ref[...] = d_ref[...] * jnp.exp(k * _LN_ALPHA)

  return pl.pallas_call(
      body, grid=(4 * nb,),
      in_specs=[pl.BlockSpec((bs, _D), lambda j: (j % nb, 0))],
      out_specs=pl.BlockSpec((bs, _D), lambda j: (j, 0)),
      out_shape=jax.ShapeDtypeStruct((4 * n, _D), jnp.float32),
  )(d)


def _leaky(x):
  return jnp.where(x >= 0.0, x, 0.01 * x)


def _tc_head(x, w1, b1, w2p, b2p, wplt, bpl, g, bet):
  n = x.shape[0]

  def body(x_ref, w1_ref, b1_ref, w2_ref, b2_ref, wp_ref, bp_ref,
           g_ref, be_ref, o_ref):
    s1 = _leaky(_mm(x_ref[...], w1_ref[...]) + b1_ref[...])
    s2 = _leaky(_mm(s1, w2_ref[...]) + b2_ref[...])
    s3 = jnp.sum(s2 * wp_ref[...], axis=1, keepdims=True) + bp_ref[...]
    mask = lax.broadcasted_iota(jnp.int32, (n, 1), 0) < _ND
    cnt = jnp.float32(_ND)
    mean = jnp.sum(jnp.where(mask, s3, 0.0)) / cnt
    dev = jnp.where(mask, s3 - mean, 0.0)
    var = jnp.sum(dev * dev) / cnt
    s = (s3 - mean) * lax.rsqrt(var + 1e-5) * g_ref[...] + be_ref[...]
    o_ref[...] = 1.0 / (1.0 + jnp.exp(-s))

  full = lambda shape: pl.BlockSpec(shape, lambda: tuple(0 for _ in shape))
  return pl.pallas_call(
      body,
      in_specs=[full((n, _D)), full((_D, _D // 2)), full((1, _D // 2)),
                full((_D // 2, _D)), full((1, _D)), full((1, _D)),
                full((1, 1)), full((1, 1)), full((1, 1))],
      out_specs=full((n, 1)),
      out_shape=jax.ShapeDtypeStruct((n, 1), jnp.float32),
  )(x, w1, b1, w2p, b2p, wplt, bpl, g, bet)


def _tc_final(ss2, r, bs):
  n = r.shape[0]

  def body(s_ref, r_ref, o1_ref, o2_ref):
    o1_ref[...] = s_ref[0, :, :1] * r_ref[...]
    o2_ref[...] = s_ref[1, :, :1] * r_ref[...]

  s32 = pl.BlockSpec((2, bs, _D), lambda i: (0, i, 0))
  sd = pl.BlockSpec((bs, _D), lambda i: (i, 0))
  sh = jax.ShapeDtypeStruct((n, _D), jnp.float32)
  return pl.pallas_call(
      body, grid=(n // bs,),
      in_specs=[s32, sd], out_specs=[sd, sd], out_shape=[sh, sh],
  )(ss2, r)


def _pad_ei(ei, dummy, chunk):
  e = ei.shape[1]
  ep = -(-e // (_NW * chunk * _GS)) * (_NW * chunk * _GS)
  src = jnp.pad(ei[0].astype(jnp.int32), (0, ep - e))
  dst = jnp.pad(ei[1].astype(jnp.int32), (0, ep - e), constant_values=dummy)
  return src.reshape(ep // chunk, chunk), dst.reshape(ep // chunk, chunk), ep


def kernel(r_node, r2r_edge, i_node, d2d_edge, r2r_ei, i2i_ei, d2d_ei, i2d_ei,
           d2r_ei, W_msg, b_msg, W_upd, b_upd, W_iP, b_iP, W_iM, b_iM, W_dP,
           b_dP, W_dM, b_dM, W1P, b1P, W2P, b2P, WplP, bplP, W1M, b1M, W2M,
           b2M, WplM, bplM, gP, betP, gM, betM):
  f32 = jnp.float32

  r = jnp.pad(r_node, ((0, _NRP - _NR), (0, 0)))
  i0 = jnp.pad(i_node, ((0, _NRP - _NI), (0, 0)))

  srcR, dstR, epR = _pad_ei(r2r_ei, _NR, 64)
  srcI, dstI, epI = _pad_ei(i2i_ei, _NI, 128)
  srcID, dstID, epID = _pad_ei(i2d_ei, _ND, 128)
  srcDD, dstDD, epDD = _pad_ei(d2d_ei, _ND, 128)
  srcDR, dstDR, epDR = _pad_ei(d2r_ei, _NR, 128)
  distp = jnp.pad(d2d_edge.astype(jnp.int32), (0, epDD - d2d_edge.shape[0]))
  srcDD = (srcDD.reshape(-1) + distp * _NDP).reshape(-1, 128)
  e16 = jnp.pad(r2r_edge, ((0, epR - r2r_edge.shape[0]), (0, 0)))

  zR = jnp.zeros((_NRP, _D), f32)
  zD = jnp.zeros((_NDP, _D), f32)

  seg_r2r = _edge_segsum(_NRP, epR, "msg", 64, nbuf=2)
  seg_i2i = _edge_segsum(_NRP, epI, "pm", 128, nbuf=2)
  seg_i2d = _edge_segsum(_NDP, epID, "pm", 128, nbuf=2)
  seg_d2d = _edge_segsum(_NDP, epDD, "pm", 128, nbuf=2)
  seg_d2r = _edge_segsum(_NRP, epDR, "pm", 128, nbuf=2)
  seg_deg = _edge_segsum(_NRP, epI, "deg", 128)

  deg2 = seg_deg(dstI, zR)

  iP = i0
  iM = i0
  for l in range(_L):
    hW = _tc_linear(r, W_msg[l, :_D, :], b_msg[l].reshape(1, -1), 1024)
    eW = _tc_linear(e16, W_msg[l, _D:, :], jnp.zeros((1, _D), f32), 2048)
    aggR = seg_r2r(hW, srcR, dstR, eW, zR)
    r = _tc_update(aggR, W_upd[l], b_upd[l].reshape(1, -1), r, 1024)
    hPin, hMin = _tc_add2(iP, iM, r, 1024)
    agg2 = seg_i2i(hPin, hMin, srcI, dstI, zR)
    iP, iM = _tc_gcn2(agg2, deg2, hPin, hMin, W_iP[l], b_iP[l].reshape(1, -1),
                      W_iM[l], b_iM[l].reshape(1, -1), 1024)

  dd2 = seg_i2d(iP, iM, srcID, dstID, zD)
  d_P, d_M = dd2[0], dd2[1]
  aggDD2 = seg_d2d(_tc_scale4(d_P, 1024), _tc_scale4(d_M, 1024),
                   srcDD, dstDD, zD)
  h_P, h_M = _tc_update2(aggDD2, W_dP, b_dP.reshape(1, -1),
                         W_dM, b_dM.reshape(1, -1), d_P, d_M, 1024)

  scores = []
  for h_d, W1, b1, W2, b2, Wpl, bpl, g, bet in (
      (h_P, W1P, b1P, W2P, b2P, WplP, bplP, gP, betP),
      (h_M, W1M, b1M, W2M, b2M, WplM, bplM, gM, betM)):
    w2p = jnp.zeros((_D // 2, _D), f32).at[:, :3].set(W2)
    b2p = jnp.zeros((1, _D), f32).at[0, :3].set(b2)
    wplt = jnp.zeros((1, _D), f32).at[0, :3].set(Wpl[:, 0])
    scores.append(_tc_head(h_d, W1, b1.reshape(1, -1), w2p, b2p, wplt,
                           bpl.reshape(1, 1), g.reshape(1, 1),
                           bet.reshape(1, 1)))
  scoreP, scoreM = scores
  ss2 = seg_d2r(jnp.pad(scoreP, ((0, 0), (0, _D - 1))),
                jnp.pad(scoreM, ((0, 0), (0, _D - 1))), srcDR, dstDR, zR)

  rP, rM = _tc_final(ss2, r, 1024)
  return (rP[:_NR], rM[:_NR], scoreP[:_ND], scoreM[:_ND])
